# Initial kernel scaffold; baseline (speedup 1.0000x reference)
import jax
import jax.numpy as jnp
from jax.experimental import pallas as pl


def kernel(src, seg, word_table, position_table, segment_table, ln_gamma, ln_beta):
    raise NotImplementedError("write your pallas kernel here")


# speedup vs baseline: 1.7890x; 1.7890x over previous
import functools

import jax
import jax.numpy as jnp
from jax import lax
from jax.experimental import pallas as pl
from jax.experimental.pallas import tpu as pltpu
from jax.experimental.pallas import tpu_sc as plsc

NUM_CORES = 2
NUM_SUBCORES = 16
NUM_WORKERS = NUM_CORES * NUM_SUBCORES
LANES = 16

VOCAB = 1000000
EMB = 64
B = 1024
L = 512
EPS = 1e-6

ROWS = B * L
ROWS_PER_WORKER = ROWS // NUM_WORKERS
BLK = 512
NBLK = ROWS_PER_WORKER // BLK


def _combine_tables_body(seg_ref, pos_ref, out_ref):
    out_ref[...] = seg_ref[...][:, None, :] + pos_ref[...][None, :, :]


def _rsqrt(x):
    i = lax.bitcast_convert_type(x, jnp.int32)
    i = jnp.int32(0x5F3759DF) - lax.shift_right_logical(i, 1)
    y = lax.bitcast_convert_type(i, jnp.float32)
    for _ in range(3):
        y = y * (jnp.float32(1.5) - jnp.float32(0.5) * x * y * y)
    return y


def _sc_body(src_hbm, seg_hbm, word_hbm, sp_hbm, gam_hbm, bet_hbm, out_hbm,
             idx_v, spidx_v, rows_v, gam_v, bet_v, sem):
    wid = lax.axis_index("s") * NUM_CORES + lax.axis_index("c")
    base = wid * ROWS_PER_WORKER

    pltpu.sync_copy(gam_hbm, gam_v)
    pltpu.sync_copy(bet_hbm, bet_v)
    g = [gam_v[pl.ds(16 * j, 16)] for j in range(4)]
    b = [bet_v[pl.ds(16 * j, 16)] for j in range(4)]

    lane = lax.iota(jnp.int32, LANES)

    def do_block(blk, carry):
        row0 = base + blk * BLK
        pltpu.sync_copy(src_hbm.at[pl.ds(row0, BLK)], idx_v)
        pltpu.sync_copy(seg_hbm.at[pl.ds(row0, BLK)], spidx_v)

        def fix_idx(i, c):
            off = i * LANES
            s = spidx_v[pl.ds(off, LANES)]
            spidx_v[pl.ds(off, LANES)] = s * jnp.int32(L) + off + lane
            return c
        lax.fori_loop(0, BLK // LANES, fix_idx, 0, unroll=4)

        pltpu.async_copy(word_hbm.at[idx_v], rows_v, sem).wait()
        pltpu.async_copy(sp_hbm.at[spidx_v], rows_v, sem, add=True).wait()

        def do_row(r, c):
            x0 = rows_v[r, pl.ds(0, 16)]
            x1 = rows_v[r, pl.ds(16, 16)]
            x2 = rows_v[r, pl.ds(32, 16)]
            x3 = rows_v[r, pl.ds(48, 16)]
            tot = jnp.sum(x0 + x1 + x2 + x3)
            totq = jnp.sum(x0 * x0 + x1 * x1 + x2 * x2 + x3 * x3)
            mean = tot * jnp.float32(1.0 / EMB)
            var = totq * jnp.float32(1.0 / EMB) - mean * mean
            rstd = _rsqrt(var + jnp.float32(EPS))
            rows_v[r, pl.ds(0, 16)] = (x0 - mean) * rstd * g[0] + b[0]
            rows_v[r, pl.ds(16, 16)] = (x1 - mean) * rstd * g[1] + b[1]
            rows_v[r, pl.ds(32, 16)] = (x2 - mean) * rstd * g[2] + b[2]
            rows_v[r, pl.ds(48, 16)] = (x3 - mean) * rstd * g[3] + b[3]
            return c
        lax.fori_loop(0, BLK, do_row, 0, unroll=4)

        pltpu.sync_copy(rows_v, out_hbm.at[pl.ds(row0, BLK)])
        return carry

    lax.fori_loop(0, NBLK, do_block, 0)


def kernel(src, seg, word_table, position_table, segment_table, ln_gamma, ln_beta):
    src_flat = src.reshape(ROWS).astype(jnp.int32)
    seg_flat = seg.reshape(ROWS).astype(jnp.int32)

    sp_table = pl.pallas_call(
        _combine_tables_body,
        out_shape=jax.ShapeDtypeStruct((3, L, EMB), jnp.float32),
    )(segment_table, position_table)
    sp_table = sp_table.reshape(3 * L, EMB)

    mesh = plsc.VectorSubcoreMesh(
        core_axis_name="c", subcore_axis_name="s",
        num_cores=NUM_CORES, num_subcores=NUM_SUBCORES)

    sc_kernel = functools.partial(
        pl.kernel,
        out_type=jax.ShapeDtypeStruct((ROWS, EMB), jnp.float32),
        mesh=mesh,
        compiler_params=pltpu.CompilerParams(
            needs_layout_passes=False, use_tc_tiling_on_sc=False),
        scratch_types=[
            pltpu.VMEM((BLK,), jnp.int32),
            pltpu.VMEM((BLK,), jnp.int32),
            pltpu.VMEM((BLK, EMB), jnp.float32),
            pltpu.VMEM((EMB,), jnp.float32),
            pltpu.VMEM((EMB,), jnp.float32),
            pltpu.SemaphoreType.DMA,
        ],
    )(_sc_body)

    out = sc_kernel(src_flat, seg_flat, word_table, sp_table, ln_gamma, ln_beta)
    return out.reshape(B, L, EMB)
